# baseline (speedup 1.0000x reference)
import functools

import jax
import jax.numpy as jnp
from jax import lax
from jax.experimental import pallas as pl
from jax.experimental.pallas import tpu as pltpu
from jax.experimental.pallas import tpu_sc as plsc

_B, _N, _M = 16, 2048, 2048
_GAMMA = 0.5
_NEG = float("-inf")
_L = _B * (_N + 1) * (_M + 1)
_ROWS = (_B * _N) // 128
_RPW = _ROWS // 32


def _sc_gather_body(sflat, g0, g1, prow, pcol, gbuf, ibuf, obuf, sem):
    wid = lax.axis_index("s") * 2 + lax.axis_index("c")
    r0 = wid * _RPW
    lane = lax.broadcasted_iota(jnp.int32, (16,), 0)
    for side in range(2):
        src = g0 if side == 0 else g1
        dst = prow if side == 0 else pcol
        pltpu.sync_copy(src.at[pl.ds(r0, _RPW)], gbuf)
        for j in range(_RPW):
            for k in range(8):
                g = gbuf[j, pl.ds(k * 16, 16)]
                p = (r0 + j) * 128 + k * 16 + lane
                bb = lax.shift_right_logical(p, 11)
                q = lax.bitwise_and(p, 2047)
                if side == 0:
                    idx = (bb * 2049 + q) * 2049 + g
                else:
                    idx = (bb * 2049 + g) * 2049 + q
                ibuf[j, pl.ds(k * 16, 16)] = idx
        copies = [
            pltpu.async_copy(sflat.at[ibuf.at[j]], obuf.at[j], sem)
            for j in range(_RPW)
        ]
        for c in copies:
            c.wait()
        pltpu.sync_copy(obuf, dst.at[pl.ds(r0, _RPW)])


_sc_gather = functools.partial(
    pl.kernel,
    out_type=[
        jax.ShapeDtypeStruct((_ROWS, 128), jnp.float32),
        jax.ShapeDtypeStruct((_ROWS, 128), jnp.float32),
    ],
    mesh=plsc.VectorSubcoreMesh(core_axis_name="c", subcore_axis_name="s"),
    scratch_types=[
        pltpu.VMEM((_RPW, 128), jnp.int32),
        pltpu.VMEM((_RPW, 128), jnp.int32),
        pltpu.VMEM((_RPW, 128), jnp.float32),
        pltpu.SemaphoreType.DMA,
    ],
)(_sc_gather_body)


def _tc_body(scores_ref, prow_ref, pcol_ref, out_ref, acc_ref):
    b = pl.program_id(0)
    s = scores_ref[0]
    rpos = prow_ref[0]
    cpos = pcol_ref[0]

    @pl.when(b == 0)
    def _():
        acc_ref[0, 0] = 0.0

    sr = s[:_N, :]
    rm1 = jnp.max(sr, axis=1, keepdims=True)
    rm2 = jnp.max(jnp.where(sr == rm1, _NEG, sr), axis=1, keepdims=True)
    neg = jnp.where(rpos == rm1, rm2, rm1)
    acc_ref[0, 0] += jnp.sum(jnp.maximum(neg - rpos + _GAMMA, 0.0))

    sc = s[:, :_M]
    cm1 = jnp.max(sc, axis=0, keepdims=True)
    cm2 = jnp.max(jnp.where(sc == cm1, _NEG, sc), axis=0, keepdims=True)
    cneg = jnp.where(cpos == cm1, cm2, cm1)
    acc_ref[0, 0] += jnp.sum(jnp.maximum(cneg - cpos + _GAMMA, 0.0))

    out_ref[...] = jnp.full((1, 1), acc_ref[0, 0] * (1.0 / (2 * _B * _N)),
                            jnp.float32)


def _tc_run(scores, prow, pcol):
    return pl.pallas_call(
        _tc_body,
        grid=(_B,),
        in_specs=[
            pl.BlockSpec((1, _N + 1, _M + 1), lambda b: (b, 0, 0)),
            pl.BlockSpec((1, _N, 1), lambda b: (b, 0, 0)),
            pl.BlockSpec((1, 1, _M), lambda b: (b, 0, 0)),
        ],
        out_specs=pl.BlockSpec((1, 1), lambda b: (0, 0)),
        out_shape=jax.ShapeDtypeStruct((1, 1), jnp.float32),
        scratch_shapes=[
            pltpu.SMEM((1, 1), jnp.float32),
        ],
    )(scores, prow, pcol)


def kernel(gt_matches0, gt_matches1, scores):
    g0 = jnp.where(gt_matches0 == -1, _M, gt_matches0).astype(jnp.int32)
    g1 = jnp.where(gt_matches1 == -1, _N, gt_matches1).astype(jnp.int32)
    sflat = scores.reshape(_L)
    prow, pcol = _sc_gather(sflat, g0.reshape(_ROWS, 128),
                            g1.reshape(_ROWS, 128))
    out = _tc_run(scores, prow.reshape(_B, _N, 1),
                  pcol.reshape(_B, 1, _M))
    return out[0, 0]
